# Initial kernel scaffold; baseline (speedup 1.0000x reference)
import jax
import jax.numpy as jnp
from jax.experimental import pallas as pl


def kernel(x, batch, W1, b1, W2):
    raise NotImplementedError("write your pallas kernel here")


# speedup vs baseline: 2.3473x; 2.3473x over previous
import functools

import jax
import jax.numpy as jnp
import numpy as np
from jax.experimental import pallas as pl

NODE_DIM = 128
N_GRAPHS = 512
N_NODES = 100000
TILE = 2000
LOG2 = float(np.log(2.0))


def _fused_body(x_ref, b_ref, w1_ref, b1_ref, w2_ref, out_ref):
    i = pl.program_id(0)

    @pl.when(i == 0)
    def _():
        out_ref[...] = jnp.zeros_like(out_ref)

    xb = x_ref[...]
    h = jnp.dot(xb, w1_ref[...], preferred_element_type=jnp.float32)
    h = h + b1_ref[...]
    h = jnp.maximum(h, 0.0) + jnp.log1p(jnp.exp(-jnp.abs(h))) - LOG2
    s = jnp.sum(h * w2_ref[...], axis=1, keepdims=True)

    seg = b_ref[0]
    gids = jax.lax.broadcasted_iota(jnp.int32, (N_GRAPHS, TILE), 0)
    onehot = (gids == seg).astype(jnp.float32)
    out_ref[...] += jnp.dot(onehot, s, preferred_element_type=jnp.float32)


@jax.jit
def _run(x, batch_r, W1, b1r, w2r):
    nb = N_NODES // TILE
    return pl.pallas_call(
        _fused_body,
        grid=(nb,),
        in_specs=[
            pl.BlockSpec((TILE, NODE_DIM), lambda i: (i, 0)),
            pl.BlockSpec((1, 1, TILE), lambda i: (i, 0, 0)),
            pl.BlockSpec((NODE_DIM, NODE_DIM), lambda i: (0, 0)),
            pl.BlockSpec((1, NODE_DIM), lambda i: (0, 0)),
            pl.BlockSpec((1, NODE_DIM), lambda i: (0, 0)),
        ],
        out_specs=pl.BlockSpec((N_GRAPHS, 1), lambda i: (0, 0)),
        out_shape=jax.ShapeDtypeStruct((N_GRAPHS, 1), jnp.float32),
    )(x, batch_r, W1, b1r, w2r)


def kernel(x, batch, W1, b1, W2):
    nb = N_NODES // TILE
    batch_r = batch.astype(jnp.int32).reshape(nb, 1, TILE)
    b1r = b1.reshape(1, NODE_DIM)
    w2r = W2.reshape(1, NODE_DIM)
    return _run(x, batch_r, W1, b1r, w2r)
